# baseline (speedup 1.0000x reference)
import jax
import jax.numpy as jnp
from jax import lax
from jax.experimental import pallas as pl
from jax.experimental.pallas import tpu as pltpu
from jax.experimental.pallas import tpu_sc as plsc

B = 16384
NODES = 1000000
K = 64
H = 128
LH = 384

NC = 2
NS = 16
NW = NC * NS
CHUNK = 2 * B // NW
IDXW = 128
NJ = CHUNK // IDXW
SH = B // NW
NG = SH // IDXW
NHALF = 4
QH = SH // NHALF
NGH = NG // NHALF

BLK1 = 4096


def _dense_body(h_ref, t_ref, wt_ref, znt_ref, b_ref, zn_ref, o_ref):
    wt = wt_ref[...]
    znt = znt_ref[...]
    zn = zn_ref[...]
    bb = b_ref[...]
    for side, x_ref in ((0, h_ref), (1, t_ref)):
        xo = jnp.dot(x_ref[...], wt, preferred_element_type=jnp.float32) + bb
        lg = jnp.dot(xo, znt, preferred_element_type=jnp.float32)
        m = jnp.max(lg, axis=-1, keepdims=True)
        e = jnp.exp(lg - m)
        probs = e / jnp.sum(e, axis=-1, keepdims=True)
        o_ref[side] = jax.nn.sigmoid(
            jnp.dot(probs, zn, preferred_element_type=jnp.float32))


def _dense_call(head_embs, tail_embs, wt, znt, b2, Zn):
    grid = (B // BLK1,)
    return pl.pallas_call(
        _dense_body,
        grid=grid,
        in_specs=[
            pl.BlockSpec((BLK1, LH), lambda i: (i, 0)),
            pl.BlockSpec((BLK1, LH), lambda i: (i, 0)),
            pl.BlockSpec((LH, H), lambda i: (0, 0)),
            pl.BlockSpec((H, K), lambda i: (0, 0)),
            pl.BlockSpec((1, H), lambda i: (0, 0)),
            pl.BlockSpec((K, H), lambda i: (0, 0)),
        ],
        out_specs=pl.BlockSpec((2, BLK1, H), lambda i: (0, i, 0)),
        out_shape=jax.ShapeDtypeStruct((2, B, H), jnp.float32),
    )(head_embs, tail_embs, wt, znt, b2, Zn)


def _sc_body(q, idxs, vals, idxg, out,
             tab, idx_v, val_v, ig_v, wh_v, wt_v, r1_v, r2_v, sem):
    c = lax.axis_index("c")
    s = lax.axis_index("s")
    wid = c * NS + s
    for side, wv in ((0, wh_v), (1, wt_v)):
        pltpu.sync_copy(idxs.at[side].at[s], idx_v)
        pltpu.sync_copy(vals.at[side].at[s], val_v)
        for j in range(NJ):
            pltpu.sync_copy(val_v.at[j], tab.at[idx_v.at[j]])
        plsc.subcore_barrier()
        pltpu.sync_copy(idxg.at[side].at[wid], ig_v)
        for j in range(NG):
            pltpu.sync_copy(tab.at[ig_v.at[j]], wv.at[j])
        plsc.subcore_barrier()
    for hh in range(NHALF):
        cps = []
        for j in range(NGH):
            cps.append(pltpu.async_copy(
                q.at[wh_v.at[hh * NGH + j]],
                r1_v.at[pl.ds(j * IDXW, IDXW)], sem))
            cps.append(pltpu.async_copy(
                q.at[wt_v.at[hh * NGH + j]],
                r2_v.at[pl.ds(j * IDXW, IDXW)], sem))
        for cp in cps:
            cp.wait()

        def _mul_row(r, _):
            for k in range(H // 16):
                sl = pl.ds(k * 16, 16)
                r1_v[r, sl] = r1_v[r, sl] * r2_v[r, sl]
            return _

        lax.fori_loop(0, QH, _mul_row, 0)
        pltpu.sync_copy(
            r1_v, out.at[pl.ds(wid * SH + hh * QH, QH)])


def _sc_call(q, idxs, vals, idxg):
    f = pl.kernel(
        _sc_body,
        out_type=jax.ShapeDtypeStruct((B, H), jnp.float32),
        mesh=plsc.VectorSubcoreMesh(
            core_axis_name="c", subcore_axis_name="s", num_cores=NC,
            num_subcores=NS),
        scratch_types=[
            pltpu.VMEM_SHARED((NODES,), jnp.int32),
            pltpu.VMEM((NJ, IDXW), jnp.int32),
            pltpu.VMEM((NJ, IDXW), jnp.int32),
            pltpu.VMEM((NG, IDXW), jnp.int32),
            pltpu.VMEM((NG, IDXW), jnp.int32),
            pltpu.VMEM((NG, IDXW), jnp.int32),
            pltpu.VMEM((QH, H), jnp.float32),
            pltpu.VMEM((QH, H), jnp.float32),
            pltpu.SemaphoreType.DMA,
        ],
    )
    return f(q, idxs, vals, idxg)


def kernel(head_embs, tail_embs, Hn, Zn, W, b, head_idx, tail_idx):
    del Hn
    wt = W.T
    znt = Zn.T
    b2 = b.reshape(1, H)
    q2 = _dense_call(head_embs, tail_embs, wt, znt, b2, Zn)
    q = q2.reshape(2 * B, H)
    both = jnp.stack([head_idx, tail_idx])
    idxs = both.reshape(2, NS, NJ, IDXW)
    vals = jnp.arange(2 * B, dtype=jnp.int32).reshape(2, NS, NJ, IDXW)
    idxg = both.reshape(2, NW, NG, IDXW)
    return _sc_call(q, idxs, vals, idxg)


# speedup vs baseline: 1.1156x; 1.1156x over previous
import jax
import jax.numpy as jnp
from jax import lax
from jax.experimental import pallas as pl
from jax.experimental.pallas import tpu as pltpu
from jax.experimental.pallas import tpu_sc as plsc

B = 16384
NODES = 1000000
K = 64
H = 128
LH = 384

NC = 2
NS = 16
NW = NC * NS
CHUNK = 2 * B // NW
IDXW = 128
NJ = CHUNK // IDXW
SH = B // NW
NG = SH // IDXW
NHALF = 2
QH = SH // NHALF
NGH = NG // NHALF

BLK1 = 4096


def _dense_body(h_ref, t_ref, wt_ref, znt_ref, b_ref, zn_ref, o_ref):
    wt = wt_ref[...]
    znt = znt_ref[...]
    zn = zn_ref[...]
    bb = b_ref[...]
    for side, x_ref in ((0, h_ref), (1, t_ref)):
        xo = jnp.dot(x_ref[...], wt, preferred_element_type=jnp.float32) + bb
        lg = jnp.dot(xo, znt, preferred_element_type=jnp.float32)
        m = jnp.max(lg, axis=-1, keepdims=True)
        e = jnp.exp(lg - m)
        probs = e / jnp.sum(e, axis=-1, keepdims=True)
        o_ref[side] = jax.nn.sigmoid(
            jnp.dot(probs, zn, preferred_element_type=jnp.float32))


def _dense_call(head_embs, tail_embs, wt, znt, b2, Zn):
    grid = (B // BLK1,)
    return pl.pallas_call(
        _dense_body,
        grid=grid,
        in_specs=[
            pl.BlockSpec((BLK1, LH), lambda i: (i, 0)),
            pl.BlockSpec((BLK1, LH), lambda i: (i, 0)),
            pl.BlockSpec((LH, H), lambda i: (0, 0)),
            pl.BlockSpec((H, K), lambda i: (0, 0)),
            pl.BlockSpec((1, H), lambda i: (0, 0)),
            pl.BlockSpec((K, H), lambda i: (0, 0)),
        ],
        out_specs=pl.BlockSpec((2, BLK1, H), lambda i: (0, i, 0)),
        out_shape=jax.ShapeDtypeStruct((2, B, H), jnp.float32),
    )(head_embs, tail_embs, wt, znt, b2, Zn)


def _dedup_body(idxs, vals, idxg, win,
                tab, idx_v, val_v, ig_v, wh_v, wt_v):
    c = lax.axis_index("c")
    s = lax.axis_index("s")
    wid = c * NS + s
    for side, wv in ((0, wh_v), (1, wt_v)):
        pltpu.sync_copy(idxs.at[side].at[s], idx_v)
        pltpu.sync_copy(vals.at[side].at[s], val_v)
        for j in range(NJ):
            pltpu.sync_copy(val_v.at[j], tab.at[idx_v.at[j]])
        plsc.subcore_barrier()
        pltpu.sync_copy(idxg.at[side].at[wid], ig_v)
        for j in range(NG):
            pltpu.sync_copy(tab.at[ig_v.at[j]], wv.at[j])
        plsc.subcore_barrier()
        pltpu.sync_copy(wv, win.at[side].at[wid])


def _dedup_call(idxs, vals, idxg):
    f = pl.kernel(
        _dedup_body,
        out_type=jax.ShapeDtypeStruct((2, NW, NG, IDXW), jnp.int32),
        mesh=plsc.VectorSubcoreMesh(
            core_axis_name="c", subcore_axis_name="s", num_cores=NC,
            num_subcores=NS),
        scratch_types=[
            pltpu.VMEM_SHARED((NODES,), jnp.int32),
            pltpu.VMEM((NJ, IDXW), jnp.int32),
            pltpu.VMEM((NJ, IDXW), jnp.int32),
            pltpu.VMEM((NG, IDXW), jnp.int32),
            pltpu.VMEM((NG, IDXW), jnp.int32),
            pltpu.VMEM((NG, IDXW), jnp.int32),
        ],
    )
    return f(idxs, vals, idxg)


def _combine_body(q, win, out, wh_v, wt_v, r1_v, r2_v, sem):
    c = lax.axis_index("c")
    s = lax.axis_index("s")
    wid = c * NS + s
    pltpu.sync_copy(win.at[0].at[wid], wh_v)
    pltpu.sync_copy(win.at[1].at[wid], wt_v)
    for hh in range(NHALF):
        cps = []
        for j in range(NGH):
            cps.append(pltpu.async_copy(
                q.at[wh_v.at[hh * NGH + j]],
                r1_v.at[pl.ds(j * IDXW, IDXW)], sem))
            cps.append(pltpu.async_copy(
                q.at[wt_v.at[hh * NGH + j]],
                r2_v.at[pl.ds(j * IDXW, IDXW)], sem))
        for cp in cps:
            cp.wait()

        def _mul_row(r, _):
            for k in range(H // 16):
                sl = pl.ds(k * 16, 16)
                r1_v[r, sl] = r1_v[r, sl] * r2_v[r, sl]
            return _

        lax.fori_loop(0, QH, _mul_row, 0)
        pltpu.sync_copy(
            r1_v, out.at[pl.ds(wid * SH + hh * QH, QH)])


def _combine_call(q, win):
    f = pl.kernel(
        _combine_body,
        out_type=jax.ShapeDtypeStruct((B, H), jnp.float32),
        mesh=plsc.VectorSubcoreMesh(
            core_axis_name="c", subcore_axis_name="s", num_cores=NC,
            num_subcores=NS),
        scratch_types=[
            pltpu.VMEM((NG, IDXW), jnp.int32),
            pltpu.VMEM((NG, IDXW), jnp.int32),
            pltpu.VMEM((QH, H), jnp.float32),
            pltpu.VMEM((QH, H), jnp.float32),
            pltpu.SemaphoreType.DMA,
        ],
    )
    return f(q, win)


def kernel(head_embs, tail_embs, Hn, Zn, W, b, head_idx, tail_idx):
    del Hn
    wt = W.T
    znt = Zn.T
    b2 = b.reshape(1, H)
    both = jnp.stack([head_idx, tail_idx])
    idxs = both.reshape(2, NS, NJ, IDXW)
    vals = jnp.arange(2 * B, dtype=jnp.int32).reshape(2, NS, NJ, IDXW)
    idxg = both.reshape(2, NW, NG, IDXW)
    win = _dedup_call(idxs, vals, idxg)
    q2 = _dense_call(head_embs, tail_embs, wt, znt, b2, Zn)
    q = q2.reshape(2 * B, H)
    return _combine_call(q, win)
